# baseline (speedup 1.0000x reference)
import functools

import jax
import jax.numpy as jnp
from jax import lax
from jax.experimental import pallas as pl
from jax.experimental.pallas import tpu as pltpu
from jax.experimental.pallas import tpu_sc as plsc

B = 4096
L = 200
V = 30522
D = 768
LANES = 16
DV = D // LANES

NC, NS = 2, 16
NW = NC * NS
ROWS_PER_W = B // NW
CHUNK_LEN = (56, 56, 48, 40)
CHUNK_OFF = (0, 56, 112, 160)
NCHUNK = len(CHUNK_LEN)
GROUPS = D // 32


_PACK_INTERLEAVES = True

V_PER_W = -(-V // NW)
CAST_CHUNK = 48


def _cast_body(table_hbm, out_hbm, bin_v, bout_v, sem):
    wid = lax.axis_index("s") * NC + lax.axis_index("c")
    row0 = wid * V_PER_W
    nrows = jnp.minimum(V - row0, V_PER_W)
    nchunks = -(-V_PER_W // CAST_CHUNK)

    def chunk_body(c, carry):
        @pl.when(c * CAST_CHUNK < nrows)
        def _():
            r0 = jnp.minimum(row0 + c * CAST_CHUNK, V - CAST_CHUNK)
            pltpu.async_copy(table_hbm.at[pl.ds(r0, CAST_CHUNK)], bin_v,
                             sem).wait()

            def conv_row(j, carry2):
                for g in range(GROUPS):
                    x0 = bin_v[j, pl.ds(g * 32, LANES)]
                    x1 = bin_v[j, pl.ds(g * 32 + LANES, LANES)]
                    bout_v[j, pl.ds(g * 32, 32)] = plsc.pack(
                        x0, x1, format=plsc.PackFormat.INTERLEAVED)
                return carry2

            lax.fori_loop(0, CAST_CHUNK, conv_row, 0)
            pltpu.sync_copy(bout_v, out_hbm.at[pl.ds(r0, CAST_CHUNK)])
        return carry

    lax.fori_loop(0, nchunks, chunk_body, 0)


@jax.jit
def _sc_cast(table):
    mesh = plsc.VectorSubcoreMesh(core_axis_name="c", subcore_axis_name="s")
    f = pl.kernel(
        _cast_body,
        out_type=jax.ShapeDtypeStruct((V, D), jnp.bfloat16),
        mesh=mesh,
        scratch_types=[
            pltpu.VMEM((CAST_CHUNK, D), jnp.float32),
            pltpu.VMEM((CAST_CHUNK, D), jnp.bfloat16),
            pltpu.SemaphoreType.DMA,
        ],
        compiler_params=pltpu.CompilerParams(
            use_tc_tiling_on_sc=False, needs_layout_passes=False),
    )
    return f(table)


def _pool_body(ids_hbm, table_hbm, out_hbm, ids_v, b0, b1, b2, b3,
               orow_v, s0, s1, s2, s3):
    bufs = (b0, b1, b2, b3)
    sems = (s0, s1, s2, s3)
    wid = lax.axis_index("s") * NC + lax.axis_index("c")
    base = wid * ROWS_PER_W
    pltpu.sync_copy(ids_hbm.at[pl.ds(base, ROWS_PER_W)], ids_v)

    def fire(r, c):
        idx = ids_v.at[r, pl.ds(CHUNK_OFF[c], CHUNK_LEN[c])]
        pltpu.async_copy(table_hbm.at[idx], bufs[c], sems[c])

    for c in range(NCHUNK):
        fire(0, c)

    idx0 = lax.iota(jnp.int32, LANES) * 2

    def row_body(r, carry):
        acc = tuple(jnp.zeros((LANES,), jnp.float32) for _ in range(DV))
        for c in range(NCHUNK):
            pltpu.make_async_copy(
                table_hbm.at[pl.ds(0, CHUNK_LEN[c])],
                bufs[c], sems[c]).wait()

            def chunk_body(j, accs, _buf=bufs[c]):
                new = []
                for g in range(GROUPS):
                    u = plsc.bitcast(_buf[j, pl.ds(g * 32, 32)], jnp.int32)
                    fe = plsc.bitcast(u << 16, jnp.float32)
                    fo = plsc.bitcast(u, jnp.float32)
                    new.append(accs[2 * g] + fe)
                    new.append(accs[2 * g + 1] + fo)
                return tuple(new)

            acc = lax.fori_loop(0, CHUNK_LEN[c], chunk_body, acc)

            @pl.when(r + 1 < ROWS_PER_W)
            def _():
                fire(r + 1, c)

        inv = jnp.float32(1.0 / L)
        for g in range(GROUPS):
            if _PACK_INTERLEAVES:
                orow_v[pl.ds(g * 32, LANES)] = acc[2 * g] * inv
                orow_v[pl.ds(g * 32 + LANES, LANES)] = acc[2 * g + 1] * inv
            else:
                plsc.store_scatter(orow_v, [idx0 + g * 32], acc[2 * g] * inv)
                plsc.store_scatter(orow_v, [idx0 + g * 32 + 1],
                                   acc[2 * g + 1] * inv)
        pltpu.sync_copy(orow_v, out_hbm.at[base + r])
        return carry

    lax.fori_loop(0, ROWS_PER_W, row_body, 0)


@functools.partial(jax.jit, donate_argnums=())
def _sc_pool(input_ids, table_bf16):
    mesh = plsc.VectorSubcoreMesh(core_axis_name="c", subcore_axis_name="s")
    f = pl.kernel(
        _pool_body,
        out_type=jax.ShapeDtypeStruct((B, D), jnp.float32),
        mesh=mesh,
        scratch_types=[
            pltpu.VMEM((ROWS_PER_W, L), jnp.int32),
            pltpu.VMEM((CHUNK_LEN[0], D), jnp.bfloat16),
            pltpu.VMEM((CHUNK_LEN[1], D), jnp.bfloat16),
            pltpu.VMEM((CHUNK_LEN[2], D), jnp.bfloat16),
            pltpu.VMEM((CHUNK_LEN[3], D), jnp.bfloat16),
            pltpu.VMEM((D,), jnp.float32),
            pltpu.SemaphoreType.DMA,
            pltpu.SemaphoreType.DMA,
            pltpu.SemaphoreType.DMA,
            pltpu.SemaphoreType.DMA,
        ],
        compiler_params=pltpu.CompilerParams(
            use_tc_tiling_on_sc=False, needs_layout_passes=False),
    )
    return f(input_ids, table_bf16)


def _head_body(x_ref, pw_ref, pb_ref, cw_ref, cb_ref, o_ref):
    x = x_ref[...]
    h = jnp.tanh(
        jnp.dot(x, pw_ref[...], preferred_element_type=jnp.float32)
        + pb_ref[...]
    )
    o_ref[...] = (
        jnp.dot(h, cw_ref[...], preferred_element_type=jnp.float32)
        + cb_ref[...]
    )


def _tc_head(pooled, pooler_w, pooler_b, cls_w, cls_b):
    cw = jnp.pad(cls_w, ((0, 0), (0, 128 - cls_w.shape[1])))
    cb = jnp.pad(cls_b, (0, 128 - cls_b.shape[0])).reshape(1, 128)
    pb = pooler_b.reshape(1, D)
    out = pl.pallas_call(
        _head_body,
        grid=(B // 512,),
        in_specs=[
            pl.BlockSpec((512, D), lambda i: (i, 0)),
            pl.BlockSpec((D, D), lambda i: (0, 0)),
            pl.BlockSpec((1, D), lambda i: (0, 0)),
            pl.BlockSpec((D, 128), lambda i: (0, 0)),
            pl.BlockSpec((1, 128), lambda i: (0, 0)),
        ],
        out_specs=pl.BlockSpec((512, 128), lambda i: (i, 0)),
        out_shape=jax.ShapeDtypeStruct((B, 128), jnp.float32),
    )(pooled, pooler_w, pb, cw, cb)
    return out[:, : cls_w.shape[1]]


def kernel(input_ids, table, pooler_w, pooler_b, cls_w, cls_b):
    pooled = _sc_pool(input_ids.astype(jnp.int32), _sc_cast(table))
    return _tc_head(pooled, pooler_w, pooler_b, cls_w, cls_b)


# speedup vs baseline: 1.0889x; 1.0889x over previous
import functools

import jax
import jax.numpy as jnp
from jax import lax
from jax.experimental import pallas as pl
from jax.experimental.pallas import tpu as pltpu
from jax.experimental.pallas import tpu_sc as plsc

B = 4096
L = 200
V = 30522
D = 768
LANES = 16
DV = D // LANES

NC, NS = 2, 16
NW = NC * NS
ROWS_PER_W = B // NW
CHUNK_LEN = (56, 56, 48, 40)
CHUNK_OFF = (0, 56, 112, 160)
NCHUNK = len(CHUNK_LEN)
GROUPS = D // 32


_PACK_INTERLEAVES = True

V_PER_W = -(-V // NW)
CAST_CHUNK = 32
CAST_NCHUNK = -(-V_PER_W // CAST_CHUNK)


def _cast_body(table_hbm, out_hbm, bi0, bi1, bo0, bo1, si0, si1, so0, so1):
    bins, bouts = (bi0, bi1), (bo0, bo1)
    sin, sout = (si0, si1), (so0, so1)
    wid = lax.axis_index("s") * NC + lax.axis_index("c")
    row0 = wid * V_PER_W

    def src(g):
        r0 = jnp.minimum(row0 + g * CAST_CHUNK, V - CAST_CHUNK)
        return r0

    pltpu.async_copy(table_hbm.at[pl.ds(src(0), CAST_CHUNK)], bins[0], sin[0])

    @pl.loop(0, CAST_NCHUNK, step=2)
    def _(c):
        for p in range(2):
            g = c + p
            @pl.when(g + 1 < CAST_NCHUNK)
            def _():
                pltpu.async_copy(
                    table_hbm.at[pl.ds(src(g + 1), CAST_CHUNK)],
                    bins[1 - p], sin[1 - p])

            pltpu.make_async_copy(
                table_hbm.at[pl.ds(0, CAST_CHUNK)], bins[p], sin[p]).wait()

            @pl.when(g >= 2)
            def _():
                pltpu.make_async_copy(
                    bouts[p], out_hbm.at[pl.ds(0, CAST_CHUNK)],
                    sout[p]).wait()

            def conv_row(j, carry2, _bi=bins[p], _bo=bouts[p]):
                for gg in range(GROUPS):
                    x0 = _bi[j, pl.ds(gg * 32, LANES)]
                    x1 = _bi[j, pl.ds(gg * 32 + LANES, LANES)]
                    _bo[j, pl.ds(gg * 32, 32)] = plsc.pack(
                        x0, x1, format=plsc.PackFormat.INTERLEAVED)
                return carry2

            lax.fori_loop(0, CAST_CHUNK, conv_row, 0)
            pltpu.async_copy(bouts[p], out_hbm.at[pl.ds(src(g), CAST_CHUNK)],
                             sout[p])

    for p in range(2):
        pltpu.make_async_copy(
            bouts[p], out_hbm.at[pl.ds(0, CAST_CHUNK)], sout[p]).wait()


@jax.jit
def _sc_cast(table):
    mesh = plsc.VectorSubcoreMesh(core_axis_name="c", subcore_axis_name="s")
    f = pl.kernel(
        _cast_body,
        out_type=jax.ShapeDtypeStruct((V, D), jnp.bfloat16),
        mesh=mesh,
        scratch_types=[
            pltpu.VMEM((CAST_CHUNK, D), jnp.float32),
            pltpu.VMEM((CAST_CHUNK, D), jnp.float32),
            pltpu.VMEM((CAST_CHUNK, D), jnp.bfloat16),
            pltpu.VMEM((CAST_CHUNK, D), jnp.bfloat16),
            pltpu.SemaphoreType.DMA,
            pltpu.SemaphoreType.DMA,
            pltpu.SemaphoreType.DMA,
            pltpu.SemaphoreType.DMA,
        ],
        compiler_params=pltpu.CompilerParams(
            use_tc_tiling_on_sc=False, needs_layout_passes=False),
    )
    return f(table)


def _pool_body(ids_hbm, table_hbm, out_hbm, ids_v, b0, b1, b2, b3,
               orow_v, s0, s1, s2, s3):
    bufs = (b0, b1, b2, b3)
    sems = (s0, s1, s2, s3)
    wid = lax.axis_index("s") * NC + lax.axis_index("c")
    base = wid * ROWS_PER_W
    pltpu.sync_copy(ids_hbm.at[pl.ds(base, ROWS_PER_W)], ids_v)

    def fire(r, c):
        idx = ids_v.at[r, pl.ds(CHUNK_OFF[c], CHUNK_LEN[c])]
        pltpu.async_copy(table_hbm.at[idx], bufs[c], sems[c])

    for c in range(NCHUNK):
        fire(0, c)

    idx0 = lax.iota(jnp.int32, LANES) * 2

    def row_body(r, carry):
        acc = tuple(jnp.zeros((LANES,), jnp.float32) for _ in range(DV))
        for c in range(NCHUNK):
            pltpu.make_async_copy(
                table_hbm.at[pl.ds(0, CHUNK_LEN[c])],
                bufs[c], sems[c]).wait()

            def chunk_body(j, accs, _buf=bufs[c]):
                new = []
                for g in range(GROUPS):
                    u = plsc.bitcast(_buf[j, pl.ds(g * 32, 32)], jnp.int32)
                    fe = plsc.bitcast(u << 16, jnp.float32)
                    fo = plsc.bitcast(u, jnp.float32)
                    new.append(accs[2 * g] + fe)
                    new.append(accs[2 * g + 1] + fo)
                return tuple(new)

            acc = lax.fori_loop(0, CHUNK_LEN[c], chunk_body, acc)

            @pl.when(r + 1 < ROWS_PER_W)
            def _():
                fire(r + 1, c)

        inv = jnp.float32(1.0 / L)
        for g in range(GROUPS):
            if _PACK_INTERLEAVES:
                orow_v[pl.ds(g * 32, LANES)] = acc[2 * g] * inv
                orow_v[pl.ds(g * 32 + LANES, LANES)] = acc[2 * g + 1] * inv
            else:
                plsc.store_scatter(orow_v, [idx0 + g * 32], acc[2 * g] * inv)
                plsc.store_scatter(orow_v, [idx0 + g * 32 + 1],
                                   acc[2 * g + 1] * inv)
        pltpu.sync_copy(orow_v, out_hbm.at[base + r])
        return carry

    lax.fori_loop(0, ROWS_PER_W, row_body, 0)


@functools.partial(jax.jit, donate_argnums=())
def _sc_pool(input_ids, table_bf16):
    mesh = plsc.VectorSubcoreMesh(core_axis_name="c", subcore_axis_name="s")
    f = pl.kernel(
        _pool_body,
        out_type=jax.ShapeDtypeStruct((B, D), jnp.float32),
        mesh=mesh,
        scratch_types=[
            pltpu.VMEM((ROWS_PER_W, L), jnp.int32),
            pltpu.VMEM((CHUNK_LEN[0], D), jnp.bfloat16),
            pltpu.VMEM((CHUNK_LEN[1], D), jnp.bfloat16),
            pltpu.VMEM((CHUNK_LEN[2], D), jnp.bfloat16),
            pltpu.VMEM((CHUNK_LEN[3], D), jnp.bfloat16),
            pltpu.VMEM((D,), jnp.float32),
            pltpu.SemaphoreType.DMA,
            pltpu.SemaphoreType.DMA,
            pltpu.SemaphoreType.DMA,
            pltpu.SemaphoreType.DMA,
        ],
        compiler_params=pltpu.CompilerParams(
            use_tc_tiling_on_sc=False, needs_layout_passes=False),
    )
    return f(input_ids, table_bf16)


def _head_body(x_ref, pw_ref, pb_ref, cw_ref, cb_ref, o_ref):
    x = x_ref[...]
    h = jnp.tanh(
        jnp.dot(x, pw_ref[...], preferred_element_type=jnp.float32)
        + pb_ref[...]
    )
    o_ref[...] = (
        jnp.dot(h, cw_ref[...], preferred_element_type=jnp.float32)
        + cb_ref[...]
    )


def _tc_head(pooled, pooler_w, pooler_b, cls_w, cls_b):
    cw = jnp.pad(cls_w, ((0, 0), (0, 128 - cls_w.shape[1])))
    cb = jnp.pad(cls_b, (0, 128 - cls_b.shape[0])).reshape(1, 128)
    pb = pooler_b.reshape(1, D)
    out = pl.pallas_call(
        _head_body,
        grid=(B // 512,),
        in_specs=[
            pl.BlockSpec((512, D), lambda i: (i, 0)),
            pl.BlockSpec((D, D), lambda i: (0, 0)),
            pl.BlockSpec((1, D), lambda i: (0, 0)),
            pl.BlockSpec((D, 128), lambda i: (0, 0)),
            pl.BlockSpec((1, 128), lambda i: (0, 0)),
        ],
        out_specs=pl.BlockSpec((512, 128), lambda i: (i, 0)),
        out_shape=jax.ShapeDtypeStruct((B, 128), jnp.float32),
    )(pooled, pooler_w, pb, cw, cb)
    return out[:, : cls_w.shape[1]]


def kernel(input_ids, table, pooler_w, pooler_b, cls_w, cls_b):
    pooled = _sc_pool(input_ids.astype(jnp.int32), _sc_cast(table))
    return _tc_head(pooled, pooler_w, pooler_b, cls_w, cls_b)
